# baseline (speedup 1.0000x reference)
import functools

import jax
import jax.numpy as jnp
from jax import lax
from jax.experimental import pallas as pl
from jax.experimental.pallas import tpu as pltpu
from jax.experimental.pallas import tpu_sc as plsc

_BATCH = 16384
_FEAT = 64
_NC = 2
_NS = 16
_NW = _NC * _NS
_SPW = _BATCH // _NW
_RPW = _SPW // 2
_CHUNK = 128
_CROWS = _CHUNK // 2
_NCHUNK = _SPW // _CHUNK
_NBUF = 3
_LANES = 16
_JCH = 128 // _LANES


def _make_kernel():
    mesh = plsc.VectorSubcoreMesh(core_axis_name="c", subcore_axis_name="s")

    @functools.partial(
        pl.kernel,
        mesh=mesh,
        out_type=jax.ShapeDtypeStruct((_NW * _LANES,), jnp.float32),
        compiler_params=pltpu.CompilerParams(use_tc_tiling_on_sc=False),
        scratch_types=[
            pltpu.VMEM((_SPW,), jnp.int32),
            pltpu.VMEM((_RPW, 128), jnp.float32),
            pltpu.VMEM((_NBUF, _CROWS, 128), jnp.float32),
            pltpu.VMEM((_LANES,), jnp.float32),
            pltpu.SemaphoreType.DMA,
            pltpu.SemaphoreType.DMA,
            pltpu.SemaphoreType.DMA,
            pltpu.SemaphoreType.DMA,
        ],
    )
    def sc_center_loss(feats_hbm, labels_hbm, centers_hbm, out_hbm,
                       idx_v, feat_v, cent_v, out_v, fsem, g0, g1, g2):
        gsems = (g0, g1, g2)
        wid = lax.axis_index("s") * _NC + lax.axis_index("c")

        pltpu.sync_copy(labels_hbm.at[pl.ds(wid * _SPW, _SPW)], idx_v)
        fcopy = pltpu.async_copy(
            feats_hbm.at[pl.ds(wid * _RPW, _RPW)], feat_v, fsem)

        def fire_chunk(c, buf):
            def fire_group(g, carry):
                idx_vec = idx_v[pl.ds(c * _CHUNK + g * _LANES, _LANES)]
                row_vec = lax.shift_right_logical(idx_vec, 1)
                col_vec = lax.shift_left(
                    lax.bitwise_and(idx_vec, 1), 6)
                for lane in range(_LANES):
                    r = row_vec[lane]
                    col = pl.multiple_of(col_vec[lane], _FEAT)
                    pltpu.async_copy(
                        centers_hbm.at[pl.ds(r, 1), pl.ds(col, _FEAT)],
                        cent_v.at[buf,
                                  pl.ds(g * (_LANES // 2) + lane // 2, 1),
                                  pl.ds((lane % 2) * _FEAT, _FEAT)],
                        gsems[buf])
                return carry
            lax.fori_loop(0, _CHUNK // _LANES, fire_group, 0)

        def drain_chunk(buf):
            pltpu.make_async_copy(
                centers_hbm.at[pl.ds(0, _CROWS)],
                cent_v.at[buf], gsems[buf]).wait()

        for b in range(_NBUF):
            fire_chunk(b, b)
        fcopy.wait()

        def row_body(c, buf):
            def body(i, accs):
                new = []
                for j in range(_JCH):
                    f = feat_v[c * _CROWS + i, pl.ds(j * _LANES, _LANES)]
                    g = cent_v[buf, i, pl.ds(j * _LANES, _LANES)]
                    d = f - g
                    new.append(accs[j] + d * d)
                return tuple(new)
            return body

        zero = jnp.zeros((_LANES,), jnp.float32)
        accs = (zero,) * _JCH
        for c in range(_NCHUNK):
            buf = c % _NBUF
            drain_chunk(buf)
            accs = lax.fori_loop(0, _CROWS, row_body(c, buf), accs)
            if c + _NBUF < _NCHUNK:
                fire_chunk(c + _NBUF, buf)

        s0 = (accs[0] + accs[1]) + (accs[2] + accs[3])
        s1 = (accs[4] + accs[5]) + (accs[6] + accs[7])
        out_v[...] = s0 + s1
        pltpu.sync_copy(out_v, out_hbm.at[pl.ds(wid * _LANES, _LANES)])

    return sc_center_loss


_sc_center_loss = _make_kernel()


def kernel(feats, labels, centers):
    feats2 = feats.reshape(_BATCH // 2, 128)
    centers2 = centers.reshape(centers.shape[0] // 2, 128)
    partials = _sc_center_loss(feats2, labels.astype(jnp.int32), centers2)
    return jnp.sum(partials) * (1.0 / _BATCH)
